# baseline (speedup 1.0000x reference)
import functools

import jax
import jax.numpy as jnp
import numpy as np
from jax import lax
from jax.experimental import pallas as pl
from jax.experimental.pallas import tpu as pltpu
from jax.experimental.pallas import tpu_sc as plsc

_RANK = 100000

_NC, _NS, _L = 2, 16, 16
_NW = _NC * _NS
_N_PER = 4 * 512 * 512
_CHUNK = _N_PER // _NW
_HALF = _CHUNK // 2


def _keys_of(b_i32):
    return b_i32 ^ ((b_i32 >> np.int32(31)) & np.int32(0x7FFFFFFF))


def _zero_hist(hist):
    @plsc.parallel_loop(0, 512, unroll=8)
    def _zero(r):
        for c in range(8):
            hist[r, pl.ds(c * _L, _L)] = jnp.zeros((_L,), jnp.int32)


def _fill_identity_idx(idx):
    for j in range(4):
        for c in range(8):
            idx[j, pl.ds(c * _L, _L)] = (
                lax.broadcasted_iota(jnp.int32, (_L,), 0)
                + np.int32(j * 128 + c * _L))


def _stream_chunks(e_hbm, m_hbm, d_hbm, base, bufs, sems, accum_fn):
    srcs = []
    for arr in (e_hbm, m_hbm, d_hbm):
        for h in range(2):
            srcs.append(arr.at[pl.ds(base + h * _HALF, _HALF)])
    copies = [None, None]
    copies[0] = pltpu.async_copy(srcs[0], bufs[0], sems[0])
    for q in range(len(srcs)):
        copies[q % 2].wait()
        if q + 1 < len(srcs):
            copies[(q + 1) % 2] = pltpu.async_copy(
                srcs[q + 1], bufs[(q + 1) % 2], sems[(q + 1) % 2])
        accum_fn(bufs[q % 2])


def _combine_into_spmem(hist, shist, idx):
    plsc.subcore_barrier()
    for j in range(4):
        pltpu.sync_copy(hist.at[pl.ds(j * 128, 128)],
                        shist.at[idx.at[j]], add=True)
    plsc.subcore_barrier()


def _suffix_scan_chunks(chunk_vals, carry, k):
    nch = len(chunk_vals)
    idx_acc = jnp.int32(0)
    s_acc = jnp.int32(0)
    iota = lax.broadcasted_iota(jnp.int32, (_L,), 0)
    for j in range(nch - 1, -1, -1):
        v = chunk_vals[j]
        c = plsc.cumsum(v)
        tot = jnp.sum(v)
        sexcl = carry + (tot - c)
        hit = (sexcl <= k) & (sexcl + v > k)
        idx_acc = idx_acc + jnp.sum(jnp.where(hit, iota + np.int32(j * _L),
                                              jnp.int32(0)))
        s_acc = s_acc + jnp.sum(jnp.where(hit, sexcl, jnp.int32(0)))
        carry = carry + tot
    return idx_acc, s_acc


def _sc_pass1(e_hbm, m_hbm, d_hbm, hist_out, rows_out,
              buf0, buf1, hist, idx, rsbuf, rstot, shist, sem0, sem1):
    cid = lax.axis_index("c")
    sid = lax.axis_index("s")
    wid = sid * _NC + cid
    base = wid * _CHUNK

    _zero_hist(hist)
    _fill_identity_idx(idx)

    @pl.when(sid == 0)
    def _():
        pltpu.sync_copy(hist, shist)

    ones = jnp.ones((_L,), jnp.int32)

    def accum(buf):
        @plsc.parallel_loop(0, _HALF // _L, unroll=8)
        def _accum(i):
            key = _keys_of(buf[pl.ds(i * _L, _L)])
            row = (key >> np.int32(23)) + np.int32(256)
            col = (key >> np.int32(16)) & np.int32(127)
            plsc.addupdate_scatter(hist, [row, col], ones)

    _stream_chunks(e_hbm, m_hbm, d_hbm, base, (buf0, buf1), (sem0, sem1),
                   accum)
    _combine_into_spmem(hist, shist, idx)

    pltpu.sync_copy(shist.at[pl.ds(sid * 32, 32)], rsbuf)
    rstot[pl.ds(0, _L)] = jnp.zeros((_L,), jnp.int32)
    rstot[pl.ds(_L, _L)] = jnp.zeros((_L,), jnp.int32)
    iota = lax.broadcasted_iota(jnp.int32, (_L,), 0)
    lane0 = iota == 0
    for r in range(32):
        acc = rsbuf[r, pl.ds(0, _L)]
        for c in range(1, 8):
            acc = acc + rsbuf[r, pl.ds(c * _L, _L)]
        s = jnp.sum(acc)
        plsc.addupdate_scatter(
            rstot, [jnp.full((_L,), np.int32(r), jnp.int32)],
            jnp.full((_L,), s, jnp.int32), mask=lane0)

    pltpu.sync_copy(rstot, rows_out.at[cid, pl.ds(sid * 32, 32)])

    @pl.when(sid == 0)
    def _():
        pltpu.sync_copy(shist, hist_out.at[cid])


def _sc_pass2(e_hbm, m_hbm, d_hbm, hist1_hbm, rows_hbm, hist_out, meta_out,
              buf0, buf1, hist, idx, rt, hrow, mvec, shist, sem0, sem1):
    cid = lax.axis_index("c")
    sid = lax.axis_index("s")
    wid = sid * _NC + cid
    base = wid * _CHUNK

    _zero_hist(hist)
    _fill_identity_idx(idx)

    @pl.when(sid == 0)
    def _():
        pltpu.sync_copy(hist, shist)

    pltpu.sync_copy(rows_hbm, rt)
    row_chunks = [rt[0, pl.ds(j * _L, _L)] + rt[1, pl.ds(j * _L, _L)]
                  for j in range(32)]
    k = np.int32(_RANK)
    rstar, sbase = _suffix_scan_chunks(row_chunks, jnp.int32(0), k)

    pltpu.sync_copy(hist1_hbm.at[0, pl.ds(rstar, 1)], hrow.at[pl.ds(0, 1)])
    pltpu.sync_copy(hist1_hbm.at[1, pl.ds(rstar, 1)], hrow.at[pl.ds(1, 1)])
    col_chunks = [hrow[0, pl.ds(j * _L, _L)] + hrow[1, pl.ds(j * _L, _L)]
                  for j in range(8)]
    cstar, sstar = _suffix_scan_chunks(col_chunks, sbase, k)

    hstar_bin = rstar * np.int32(128) + cstar
    kprime = k - sstar
    hs = jnp.full((_L,), hstar_bin - np.int32(32768), jnp.int32)

    ones = jnp.ones((_L,), jnp.int32)

    def accum(buf):
        @plsc.parallel_loop(0, _HALF // _L, unroll=8)
        def _accum(i):
            key = _keys_of(buf[pl.ds(i * _L, _L)])
            row = (key >> np.int32(7)) & np.int32(511)
            col = key & np.int32(127)
            plsc.addupdate_scatter(hist, [row, col], ones,
                                   mask=(key >> np.int32(16)) == hs)

    _stream_chunks(e_hbm, m_hbm, d_hbm, base, (buf0, buf1), (sem0, sem1),
                   accum)
    _combine_into_spmem(hist, shist, idx)

    iota = lax.broadcasted_iota(jnp.int32, (_L,), 0)
    mvec[...] = jnp.where(iota == 0, hstar_bin,
                          jnp.where(iota == 1, kprime, jnp.int32(0)))

    @pl.when(wid == 0)
    def _():
        pltpu.sync_copy(mvec, meta_out)

    @pl.when(sid == 0)
    def _():
        pltpu.sync_copy(shist, hist_out.at[cid])


def _make_sc_pass1():
    mesh = plsc.VectorSubcoreMesh(core_axis_name="c", subcore_axis_name="s",
                                  num_cores=_NC, num_subcores=_NS)
    return pl.kernel(
        _sc_pass1,
        out_type=(jax.ShapeDtypeStruct((_NC, 512, 128), jnp.int32),
                  jax.ShapeDtypeStruct((_NC, 512), jnp.int32)),
        mesh=mesh,
        compiler_params=pltpu.CompilerParams(needs_layout_passes=False),
        scratch_types=[
            pltpu.VMEM((_HALF,), jnp.int32),
            pltpu.VMEM((_HALF,), jnp.int32),
            pltpu.VMEM((512, 128), jnp.int32),
            pltpu.VMEM((4, 128), jnp.int32),
            pltpu.VMEM((32, 128), jnp.int32),
            pltpu.VMEM((32,), jnp.int32),
            pltpu.VMEM_SHARED((512, 128), jnp.int32),
            pltpu.SemaphoreType.DMA,
            pltpu.SemaphoreType.DMA,
        ],
    )


def _make_sc_pass2():
    mesh = plsc.VectorSubcoreMesh(core_axis_name="c", subcore_axis_name="s",
                                  num_cores=_NC, num_subcores=_NS)
    return pl.kernel(
        _sc_pass2,
        out_type=(jax.ShapeDtypeStruct((_NC, 512, 128), jnp.int32),
                  jax.ShapeDtypeStruct((_L,), jnp.int32)),
        mesh=mesh,
        compiler_params=pltpu.CompilerParams(needs_layout_passes=False),
        scratch_types=[
            pltpu.VMEM((_HALF,), jnp.int32),
            pltpu.VMEM((_HALF,), jnp.int32),
            pltpu.VMEM((512, 128), jnp.int32),
            pltpu.VMEM((4, 128), jnp.int32),
            pltpu.VMEM((2, 512), jnp.int32),
            pltpu.VMEM((2, 128), jnp.int32),
            pltpu.VMEM((_L,), jnp.int32),
            pltpu.VMEM_SHARED((512, 128), jnp.int32),
            pltpu.SemaphoreType.DMA,
            pltpu.SemaphoreType.DMA,
        ],
    )


def _search_hist(hists_i32, k_f32):
    h = jnp.sum(hists_i32.astype(jnp.float32), axis=0)
    tri512 = (lax.broadcasted_iota(jnp.int32, (512, 512), 0)
              > lax.broadcasted_iota(jnp.int32, (512, 512), 1)
              ).astype(jnp.float32)
    tri128 = (lax.broadcasted_iota(jnp.int32, (128, 128), 0)
              > lax.broadcasted_iota(jnp.int32, (128, 128), 1)
              ).astype(jnp.float32)

    dot = functools.partial(jnp.dot, precision=lax.Precision.HIGHEST,
                            preferred_element_type=jnp.float32)
    rsum = jnp.sum(h, axis=1)[None, :]
    srow = dot(rsum, tri512)
    row_hit = ((srow <= k_f32) & (srow + rsum > k_f32)).astype(jnp.float32)
    iota_row = lax.broadcasted_iota(jnp.int32, (1, 512), 1).astype(jnp.float32)
    rstar = jnp.sum(row_hit * iota_row)
    sbase = jnp.sum(row_hit * srow)

    colvec = dot(row_hit, h)
    scol = dot(colvec, tri128)
    tot = sbase + scol
    col_hit = ((tot <= k_f32) & (tot + colvec > k_f32)).astype(jnp.float32)
    iota_col = lax.broadcasted_iota(jnp.int32, (1, 128), 1).astype(jnp.float32)
    cstar = jnp.sum(col_hit * iota_col)
    sstar = sbase + jnp.sum(col_hit * scol)
    return rstar * 128.0 + cstar, sstar


def _tc_thresh_mask_body(hists_ref, meta_ref, e_ref, m_ref, d_ref,
                         oe_ref, om_ref, od_ref):
    hstar = meta_ref[0]
    kprime = meta_ref[1].astype(jnp.float32)
    lowstar, _ = _search_hist(hists_ref[...], kprime)
    t = ((hstar - np.int32(32768)) << np.int32(16)) | lowstar.astype(jnp.int32)
    bits = t ^ ((t >> np.int32(31)) & np.int32(0x7FFFFFFF))
    thresh = lax.bitcast_convert_type(bits, jnp.float32)
    for src, dst in ((e_ref, oe_ref), (m_ref, om_ref), (d_ref, od_ref)):
        x = src[...]
        dst[...] = jnp.where(x < thresh, jnp.float32(0.0), x)


def kernel(early, middle, deep):
    shp = early.shape
    eb = lax.bitcast_convert_type(early, jnp.int32).reshape(-1)
    mb = lax.bitcast_convert_type(middle, jnp.int32).reshape(-1)
    db = lax.bitcast_convert_type(deep, jnp.int32).reshape(-1)

    hist1, rows1 = _make_sc_pass1()(eb, mb, db)
    hist2, meta = _make_sc_pass2()(eb, mb, db, hist1, rows1)

    oe, om, od = pl.pallas_call(
        _tc_thresh_mask_body,
        out_shape=tuple(
            jax.ShapeDtypeStruct((1024, 1024), jnp.float32) for _ in range(3)),
    )(hist2, meta, early.reshape(1024, 1024), middle.reshape(1024, 1024),
      deep.reshape(1024, 1024))

    return (oe.reshape(shp), om.reshape(shp), od.reshape(shp))


# speedup vs baseline: 1.0310x; 1.0310x over previous
import functools

import jax
import jax.numpy as jnp
import numpy as np
from jax import lax
from jax.experimental import pallas as pl
from jax.experimental.pallas import tpu as pltpu
from jax.experimental.pallas import tpu_sc as plsc

_RANK = 100000

_NC, _NS, _L = 2, 16, 16
_NW = _NC * _NS
_N_PER = 4 * 512 * 512
_CHUNK = _N_PER // _NW
_HALF = _CHUNK // 2
_NBINS = 65536


def _keys_of(b_i32):
    return b_i32 ^ ((b_i32 >> np.int32(31)) & np.int32(0x7FFFFFFF))


def _sc_hist_common(e_hbm, m_hbm, d_hbm, meta_hbm, out_hbm,
                    buf0, buf1, hist, hvec, idx, shist, sem0, sem1,
                    *, low_pass):
    cid = lax.axis_index("c")
    sid = lax.axis_index("s")
    wid = sid * _NC + cid
    base = wid * _CHUNK

    @plsc.parallel_loop(0, 512, unroll=8)
    def _zero(r):
        for c in range(8):
            hist[r, pl.ds(c * _L, _L)] = jnp.zeros((_L,), jnp.int32)

    for j in range(4):
        for c in range(8):
            idx[j, pl.ds(c * _L, _L)] = (
                lax.broadcasted_iota(jnp.int32, (_L,), 0)
                + np.int32(j * 128 + c * _L))

    @pl.when(sid == 0)
    def _():
        pltpu.sync_copy(hist, shist)

    if low_pass:
        pltpu.sync_copy(meta_hbm.at[pl.ds(0, _L)], hvec)
        hs = hvec[...] - np.int32(32768)
    ones = jnp.ones((_L,), jnp.int32)

    bufs = (buf0, buf1)
    sems = (sem0, sem1)
    srcs = []
    for arr in (e_hbm, m_hbm, d_hbm):
        for h in range(2):
            srcs.append(arr.at[pl.ds(base + h * _HALF, _HALF)])

    copies = [None, None]
    copies[0] = pltpu.async_copy(srcs[0], bufs[0], sems[0])
    for q in range(len(srcs)):
        copies[q % 2].wait()
        if q + 1 < len(srcs):
            copies[(q + 1) % 2] = pltpu.async_copy(
                srcs[q + 1], bufs[(q + 1) % 2], sems[(q + 1) % 2])
        buf = bufs[q % 2]

        @plsc.parallel_loop(0, _HALF // _L, unroll=16)
        def _accum(i):
            key = _keys_of(buf[pl.ds(i * _L, _L)])
            if low_pass:
                row = (key >> np.int32(7)) & np.int32(511)
                col = key & np.int32(127)
                plsc.addupdate_scatter(
                    hist, [row, col], ones,
                    mask=(key >> np.int32(16)) == hs)
            else:
                row = (key >> np.int32(23)) + np.int32(256)
                col = (key >> np.int32(16)) & np.int32(127)
                plsc.addupdate_scatter(hist, [row, col], ones)

    plsc.subcore_barrier()
    for j in range(4):
        pltpu.sync_copy(hist.at[pl.ds(j * 128, 128)],
                        shist.at[idx.at[j]], add=True)
    plsc.subcore_barrier()

    @pl.when(sid == 0)
    def _():
        pltpu.sync_copy(shist, out_hbm.at[cid])


def _make_sc_hist(low_pass):
    mesh = plsc.VectorSubcoreMesh(core_axis_name="c", subcore_axis_name="s",
                                  num_cores=_NC, num_subcores=_NS)
    return pl.kernel(
        functools.partial(_sc_hist_common, low_pass=low_pass),
        out_type=jax.ShapeDtypeStruct((_NC, 512, 128), jnp.int32),
        mesh=mesh,
        compiler_params=pltpu.CompilerParams(needs_layout_passes=False),
        scratch_types=[
            pltpu.VMEM((_HALF,), jnp.int32),
            pltpu.VMEM((_HALF,), jnp.int32),
            pltpu.VMEM((512, 128), jnp.int32),
            pltpu.VMEM((_L,), jnp.int32),
            pltpu.VMEM((4, 128), jnp.int32),
            pltpu.VMEM_SHARED((512, 128), jnp.int32),
            pltpu.SemaphoreType.DMA,
            pltpu.SemaphoreType.DMA,
        ],
    )


def _search_hist(hists_i32, k_f32):
    h = jnp.sum(hists_i32.astype(jnp.float32), axis=0)
    tri512 = (lax.broadcasted_iota(jnp.int32, (512, 512), 0)
              > lax.broadcasted_iota(jnp.int32, (512, 512), 1)
              ).astype(jnp.float32)
    tri128 = (lax.broadcasted_iota(jnp.int32, (128, 128), 0)
              > lax.broadcasted_iota(jnp.int32, (128, 128), 1)
              ).astype(jnp.float32)

    dot = functools.partial(jnp.dot, precision=lax.Precision.HIGHEST,
                            preferred_element_type=jnp.float32)
    rsum = jnp.sum(h, axis=1)[None, :]
    srow = dot(rsum, tri512)
    row_hit = ((srow <= k_f32) & (srow + rsum > k_f32)).astype(jnp.float32)
    iota_row = lax.broadcasted_iota(jnp.int32, (1, 512), 1).astype(jnp.float32)
    rstar = jnp.sum(row_hit * iota_row)
    sbase = jnp.sum(row_hit * srow)

    colvec = dot(row_hit, h)
    scol = dot(colvec, tri128)
    tot = sbase + scol
    col_hit = ((tot <= k_f32) & (tot + colvec > k_f32)).astype(jnp.float32)
    iota_col = lax.broadcasted_iota(jnp.int32, (1, 128), 1).astype(jnp.float32)
    cstar = jnp.sum(col_hit * iota_col)
    sstar = sbase + jnp.sum(col_hit * scol)
    return rstar * 128.0 + cstar, sstar


def _tc_find_bin_body(hists_ref, meta_ref):
    hstar, sstar = _search_hist(hists_ref[...], jnp.float32(_RANK))
    kprime = jnp.float32(_RANK) - sstar
    row = lax.broadcasted_iota(jnp.int32, (8, 128), 0)
    meta_ref[...] = jnp.where(row == 0, hstar.astype(jnp.int32),
                              kprime.astype(jnp.int32))


def _tc_thresh_mask_body(hists_ref, meta_ref, e_ref, m_ref, d_ref,
                         oe_ref, om_ref, od_ref):
    kprime = meta_ref[1, 0].astype(jnp.float32)
    hstar = meta_ref[0, 0]
    lowstar, _ = _search_hist(hists_ref[...], kprime)
    t = ((hstar - np.int32(32768)) << np.int32(16)) | lowstar.astype(jnp.int32)
    bits = t ^ ((t >> np.int32(31)) & np.int32(0x7FFFFFFF))
    thresh = lax.bitcast_convert_type(bits, jnp.float32)
    for src, dst in ((e_ref, oe_ref), (m_ref, om_ref), (d_ref, od_ref)):
        x = src[...]
        dst[...] = jnp.where(x < thresh, jnp.float32(0.0), x)


def kernel(early, middle, deep):
    shp = early.shape
    eb = lax.bitcast_convert_type(early, jnp.int32).reshape(-1)
    mb = lax.bitcast_convert_type(middle, jnp.int32).reshape(-1)
    db = lax.bitcast_convert_type(deep, jnp.int32).reshape(-1)
    unused_meta = jnp.zeros((1024,), jnp.int32)

    hist1 = _make_sc_hist(low_pass=False)(eb, mb, db, unused_meta)

    meta = pl.pallas_call(
        _tc_find_bin_body,
        out_shape=jax.ShapeDtypeStruct((8, 128), jnp.int32),
    )(hist1)

    hist2 = _make_sc_hist(low_pass=True)(eb, mb, db, meta.reshape(-1))

    oe, om, od = pl.pallas_call(
        _tc_thresh_mask_body,
        out_shape=tuple(
            jax.ShapeDtypeStruct((1024, 1024), jnp.float32) for _ in range(3)),
    )(hist2, meta, early.reshape(1024, 1024), middle.reshape(1024, 1024),
      deep.reshape(1024, 1024))

    return (oe.reshape(shp), om.reshape(shp), od.reshape(shp))


# speedup vs baseline: 1.0383x; 1.0071x over previous
import functools

import jax
import jax.numpy as jnp
import numpy as np
from jax import lax
from jax.experimental import pallas as pl
from jax.experimental.pallas import tpu as pltpu
from jax.experimental.pallas import tpu_sc as plsc

_RANK = 100000

_NC, _NS, _L = 2, 16, 16
_NW = _NC * _NS
_N_PER = 4 * 512 * 512
_CHUNK = _N_PER // _NW
_HALF = _CHUNK // 2
_NBINS = 65536


def _keys_of(b_i32):
    return b_i32 ^ ((b_i32 >> np.int32(31)) & np.int32(0x7FFFFFFF))


def _sc_hist_common(e_hbm, m_hbm, d_hbm, meta_hbm, out_hbm,
                    buf0, buf1, hist, hvec, idx, shist, sem0, sem1,
                    *, low_pass):
    cid = lax.axis_index("c")
    sid = lax.axis_index("s")
    wid = sid * _NC + cid
    base = wid * _CHUNK

    @plsc.parallel_loop(0, 512, unroll=8)
    def _zero(r):
        for c in range(8):
            hist[r, pl.ds(c * _L, _L)] = jnp.zeros((_L,), jnp.int32)

    for j in range(4):
        for c in range(8):
            idx[j, pl.ds(c * _L, _L)] = (
                lax.broadcasted_iota(jnp.int32, (_L,), 0)
                + np.int32(j * 128 + c * _L))

    @pl.when(sid == 0)
    def _():
        pltpu.sync_copy(hist, shist)

    if low_pass:
        pltpu.sync_copy(meta_hbm.at[pl.ds(0, _L)], hvec)
        hs = hvec[...] - np.int32(32768)
    ones = jnp.ones((_L,), jnp.int32)

    bufs = (buf0, buf1)
    sems = (sem0, sem1)
    srcs = []
    for arr in (e_hbm, m_hbm, d_hbm):
        for h in range(2):
            srcs.append(arr.at[pl.ds(base + h * _HALF, _HALF)])

    copies = [None, None]
    copies[0] = pltpu.async_copy(srcs[0], bufs[0], sems[0])
    for q in range(len(srcs)):
        copies[q % 2].wait()
        if q + 1 < len(srcs):
            copies[(q + 1) % 2] = pltpu.async_copy(
                srcs[q + 1], bufs[(q + 1) % 2], sems[(q + 1) % 2])
        buf = bufs[q % 2]

        @plsc.parallel_loop(0, _HALF // _L, unroll=8)
        def _accum(i):
            key = _keys_of(buf[pl.ds(i * _L, _L)])
            if low_pass:
                row = (key >> np.int32(7)) & np.int32(511)
                col = key & np.int32(127)
                plsc.addupdate_scatter(
                    hist, [row, col], ones,
                    mask=(key >> np.int32(16)) == hs)
            else:
                row = (key >> np.int32(23)) + np.int32(256)
                col = (key >> np.int32(16)) & np.int32(127)
                plsc.addupdate_scatter(hist, [row, col], ones)

    plsc.subcore_barrier()
    for j in range(4):
        pltpu.sync_copy(hist.at[pl.ds(j * 128, 128)],
                        shist.at[idx.at[j]], add=True)
    plsc.subcore_barrier()

    @pl.when(sid == 0)
    def _():
        pltpu.sync_copy(shist, out_hbm.at[cid])


def _make_sc_hist(low_pass):
    mesh = plsc.VectorSubcoreMesh(core_axis_name="c", subcore_axis_name="s",
                                  num_cores=_NC, num_subcores=_NS)
    return pl.kernel(
        functools.partial(_sc_hist_common, low_pass=low_pass),
        out_type=jax.ShapeDtypeStruct((_NC, 512, 128), jnp.int32),
        mesh=mesh,
        compiler_params=pltpu.CompilerParams(needs_layout_passes=False),
        scratch_types=[
            pltpu.VMEM((_HALF,), jnp.int32),
            pltpu.VMEM((_HALF,), jnp.int32),
            pltpu.VMEM((512, 128), jnp.int32),
            pltpu.VMEM((_L,), jnp.int32),
            pltpu.VMEM((4, 128), jnp.int32),
            pltpu.VMEM_SHARED((512, 128), jnp.int32),
            pltpu.SemaphoreType.DMA,
            pltpu.SemaphoreType.DMA,
        ],
    )


def _search_hist(hists_i32, k_f32):
    h = jnp.sum(hists_i32.astype(jnp.float32), axis=0)
    tri512 = (lax.broadcasted_iota(jnp.int32, (512, 512), 0)
              > lax.broadcasted_iota(jnp.int32, (512, 512), 1)
              ).astype(jnp.float32)
    tri128 = (lax.broadcasted_iota(jnp.int32, (128, 128), 0)
              > lax.broadcasted_iota(jnp.int32, (128, 128), 1)
              ).astype(jnp.float32)

    dot = functools.partial(jnp.dot, precision=lax.Precision.HIGHEST,
                            preferred_element_type=jnp.float32)
    rsum = jnp.sum(h, axis=1)[None, :]
    srow = dot(rsum, tri512)
    row_hit = ((srow <= k_f32) & (srow + rsum > k_f32)).astype(jnp.float32)
    iota_row = lax.broadcasted_iota(jnp.int32, (1, 512), 1).astype(jnp.float32)
    rstar = jnp.sum(row_hit * iota_row)
    sbase = jnp.sum(row_hit * srow)

    colvec = dot(row_hit, h)
    scol = dot(colvec, tri128)
    tot = sbase + scol
    col_hit = ((tot <= k_f32) & (tot + colvec > k_f32)).astype(jnp.float32)
    iota_col = lax.broadcasted_iota(jnp.int32, (1, 128), 1).astype(jnp.float32)
    cstar = jnp.sum(col_hit * iota_col)
    sstar = sbase + jnp.sum(col_hit * scol)
    return rstar * 128.0 + cstar, sstar


def _tc_find_bin_body(hists_ref, meta_ref):
    hstar, sstar = _search_hist(hists_ref[...], jnp.float32(_RANK))
    kprime = jnp.float32(_RANK) - sstar
    row = lax.broadcasted_iota(jnp.int32, (8, 128), 0)
    meta_ref[...] = jnp.where(row == 0, hstar.astype(jnp.int32),
                              kprime.astype(jnp.int32))


def _tc_thresh_mask_body(hists_ref, meta_ref, e_ref, m_ref, d_ref,
                         oe_ref, om_ref, od_ref):
    kprime = meta_ref[1, 0].astype(jnp.float32)
    hstar = meta_ref[0, 0]
    lowstar, _ = _search_hist(hists_ref[...], kprime)
    t = ((hstar - np.int32(32768)) << np.int32(16)) | lowstar.astype(jnp.int32)
    bits = t ^ ((t >> np.int32(31)) & np.int32(0x7FFFFFFF))
    thresh = lax.bitcast_convert_type(bits, jnp.float32)
    for src, dst in ((e_ref, oe_ref), (m_ref, om_ref), (d_ref, od_ref)):
        x = src[...]
        dst[...] = jnp.where(x < thresh, jnp.float32(0.0), x)


def kernel(early, middle, deep):
    shp = early.shape
    eb = lax.bitcast_convert_type(early, jnp.int32).reshape(-1)
    mb = lax.bitcast_convert_type(middle, jnp.int32).reshape(-1)
    db = lax.bitcast_convert_type(deep, jnp.int32).reshape(-1)
    unused_meta = jnp.zeros((1024,), jnp.int32)

    hist1 = _make_sc_hist(low_pass=False)(eb, mb, db, unused_meta)

    meta = pl.pallas_call(
        _tc_find_bin_body,
        out_shape=jax.ShapeDtypeStruct((8, 128), jnp.int32),
    )(hist1)

    hist2 = _make_sc_hist(low_pass=True)(eb, mb, db, meta.reshape(-1))

    oe, om, od = pl.pallas_call(
        _tc_thresh_mask_body,
        out_shape=tuple(
            jax.ShapeDtypeStruct((1024, 1024), jnp.float32) for _ in range(3)),
    )(hist2, meta, early.reshape(1024, 1024), middle.reshape(1024, 1024),
      deep.reshape(1024, 1024))

    return (oe.reshape(shp), om.reshape(shp), od.reshape(shp))
